# Initial kernel scaffold; baseline (speedup 1.0000x reference)
import jax
import jax.numpy as jnp
from jax.experimental import pallas as pl


def kernel(x, edge_index, W_gat, att_src, att_dst, b_gat, W_gcn, b_gcn, W_gate, b_gate, gamma, beta):
    raise NotImplementedError("write your pallas kernel here")


# speedup vs baseline: 11.5887x; 11.5887x over previous
import functools
import jax
import jax.numpy as jnp
from jax import lax
from jax.experimental import pallas as pl
from jax.experimental.pallas import tpu as pltpu
from jax.experimental.pallas import tpu_sc as plsc

N_NODES = 10000
NPAD = 10240
N_EDGES = 320000
D = 128
H = 4
C = 32

NC = 2
NS = 16
EPT = N_EDGES // NS
CHUNK = 80
NCHUNK = EPT // CHUNK
NPT = NPAD // NS


def _tc_pre_body(x_ref, wg_ref, asrc_ref, adst_ref, r_ref, wgcn_ref,
                 xw_ref, asd_ref, exself_ref, initgat_ref, xg_ref):
    x = x_ref[...]
    xw = jnp.dot(x, wg_ref[...], preferred_element_type=jnp.float32)
    a_s = jnp.dot(xw, asrc_ref[...], preferred_element_type=jnp.float32)
    a_d = jnp.dot(xw, adst_ref[...], preferred_element_type=jnp.float32)
    al = a_s + a_d
    al = jnp.where(al > 0, al, 0.2 * al)
    ex = jnp.exp(al)
    xw_ref[...] = xw
    asd_ref[...] = jnp.concatenate([a_s, a_d], axis=1)
    exself_ref[...] = jnp.concatenate([ex, jnp.zeros_like(ex)], axis=1)
    initgat_ref[...] = jnp.dot(ex, r_ref[...], preferred_element_type=jnp.float32) * xw
    xg_ref[...] = jnp.dot(x, wgcn_ref[...], preferred_element_type=jnp.float32)


def _tc_post_body(num_ref, den_ref, u_ref, deg_ref, xg_ref, x_ref, r_ref,
                  w0_ref, w1_ref, bgate_ref, bgat_ref, bgcn_ref, g_ref, b_ref,
                  out_ref):
    den4 = den_ref[:, :4]
    denb = jnp.dot(den4 + 1e-16, r_ref[...], preferred_element_type=jnp.float32)
    gat = num_ref[...] / denb + bgat_ref[...]
    deg = deg_ref[:, 0:1] + 1.0
    dis = lax.rsqrt(deg)
    xg = xg_ref[...]
    gcn = dis * u_ref[...] + (dis * dis) * xg + bgcn_ref[...]
    z0 = (jnp.dot(gat, w0_ref[:, 0:1], preferred_element_type=jnp.float32)
          + jnp.dot(gcn, w1_ref[:, 0:1], preferred_element_type=jnp.float32)
          + bgate_ref[0, 0])
    z1 = (jnp.dot(gat, w0_ref[:, 1:2], preferred_element_type=jnp.float32)
          + jnp.dot(gcn, w1_ref[:, 1:2], preferred_element_type=jnp.float32)
          + bgate_ref[0, 1])
    gw0 = 1.0 / (1.0 + jnp.exp(z1 - z0))
    gw1 = 1.0 - gw0
    y = gw0 * gat + gw1 * gcn + x_ref[...]
    mu = jnp.mean(y, axis=-1, keepdims=True)
    yc = y - mu
    var = jnp.mean(yc * yc, axis=-1, keepdims=True)
    out_ref[...] = g_ref[...] * yc * lax.rsqrt(var + 1e-5) + b_ref[...]


def _sc_body(src_hbm, dst_hbm, asd_hbm, xw_hbm, xg_hbm, exself_hbm,
             initgat_hbm, z128_hbm, z8_hbm,
             num_out, den_out, u_out, deg_out,
             acc_s, den_s, dis_s,
             rows_v, asrc_v, adst_v, exbuf_v, ones_v, dis_v, deg_v, dis640_v,
             src_v, dst_v, sem):
    core = lax.axis_index("c")
    tile = lax.axis_index("s")
    r0 = tile * NPT
    lanes0 = lax.iota(jnp.int32, 16)

    def full16(v):
        return jnp.full((16,), v, jnp.int32)

    @pl.when(core == 0)
    def _gat():
        pltpu.sync_copy(initgat_hbm.at[pl.ds(r0, NPT)], acc_s.at[pl.ds(r0, NPT)])
        pltpu.sync_copy(exself_hbm.at[pl.ds(r0, NPT)], den_s.at[pl.ds(r0, NPT)])
        def _zero(k, carry):
            for col in range(4, 8):
                plsc.store_scatter(exbuf_v, [lanes0 + k * 16, full16(col)],
                                   jnp.zeros((16,), jnp.float32))
            return carry
        lax.fori_loop(0, CHUNK // 16, _zero, 0)
        plsc.subcore_barrier()

        def _chunk(i, carry):
            off = tile * EPT + i * CHUNK
            pltpu.sync_copy(src_hbm.at[pl.ds(off, CHUNK)], src_v)
            pltpu.sync_copy(dst_hbm.at[pl.ds(off, CHUNK)], dst_v)
            cg = pltpu.async_copy(xw_hbm.at[src_v], rows_v, sem)
            ca = pltpu.async_copy(asd_hbm.at[src_v], asrc_v, sem)
            cb = pltpu.async_copy(asd_hbm.at[dst_v], adst_v, sem)
            cg.wait()
            ca.wait()
            cb.wait()
            for g in range(CHUNK // 16):
                lane = lanes0 + g * 16
                exs = []
                for h in range(H):
                    av = plsc.load_gather(asrc_v, [lane, full16(h)])
                    bv = plsc.load_gather(adst_v, [lane, full16(h + 4)])
                    al = av + bv
                    al = jnp.where(al > 0, al, 0.2 * al)
                    e = jnp.exp(al)
                    exs.append(e)
                    plsc.store_scatter(exbuf_v, [lane, full16(h)], e)
                for c in range(D):
                    col = plsc.load_gather(rows_v, [lane, full16(c)])
                    plsc.store_scatter(rows_v, [lane, full16(c)],
                                       col * exs[c // C])
            pltpu.sync_copy(rows_v, acc_s.at[dst_v], add=True)
            pltpu.sync_copy(exbuf_v, den_s.at[dst_v], add=True)
            return carry
        lax.fori_loop(0, NCHUNK, _chunk, 0)
        plsc.subcore_barrier()
        pltpu.sync_copy(acc_s.at[pl.ds(r0, NPT)], num_out.at[pl.ds(r0, NPT)])
        pltpu.sync_copy(den_s.at[pl.ds(r0, NPT)], den_out.at[pl.ds(r0, NPT)])

    @pl.when(core == 1)
    def _gcn():
        pltpu.sync_copy(z128_hbm.at[pl.ds(r0, NPT)], acc_s.at[pl.ds(r0, NPT)])
        pltpu.sync_copy(z8_hbm.at[pl.ds(r0, NPT)], den_s.at[pl.ds(r0, NPT)])
        def _ones(k, carry):
            for col in range(8):
                plsc.store_scatter(ones_v, [lanes0 + k * 16, full16(col)],
                                   jnp.ones((16,), jnp.float32))
            return carry
        lax.fori_loop(0, CHUNK // 16, _ones, 0)
        plsc.subcore_barrier()

        def _dchunk(i, carry):
            off = tile * EPT + i * CHUNK
            pltpu.sync_copy(dst_hbm.at[pl.ds(off, CHUNK)], dst_v)
            pltpu.sync_copy(ones_v, den_s.at[dst_v], add=True)
            return carry
        lax.fori_loop(0, NCHUNK, _dchunk, 0)
        plsc.subcore_barrier()
        @pl.when(tile == 0)
        def _wdeg():
            pltpu.sync_copy(den_s, deg_out)

        pltpu.sync_copy(den_s.at[pl.ds(r0, NPT)], deg_v)
        def _newton(j, carry):
            idx = lanes0 + j * 16
            dv = plsc.load_gather(deg_v, [idx, full16(0)])
            xx = dv + 1.0
            ii = plsc.bitcast(xx, jnp.int32)
            ii = jnp.int32(0x5F3759DF) - (ii >> 1)
            y = plsc.bitcast(ii, jnp.float32)
            for _ in range(3):
                y = y * (1.5 - (0.5 * xx) * (y * y))
            plsc.store_scatter(dis640_v, [idx], y)
            return carry
        lax.fori_loop(0, NPT // 16, _newton, 0)
        pltpu.sync_copy(dis640_v, dis_s.at[pl.ds(r0, NPT)])
        plsc.subcore_barrier()
        pltpu.sync_copy(dis_s, dis_v)

        def _chunk(i, carry):
            off = tile * EPT + i * CHUNK
            pltpu.sync_copy(src_hbm.at[pl.ds(off, CHUNK)], src_v)
            pltpu.sync_copy(dst_hbm.at[pl.ds(off, CHUNK)], dst_v)
            pltpu.async_copy(xg_hbm.at[src_v], rows_v, sem).wait()
            for g in range(CHUNK // 16):
                s16 = src_v[pl.ds(g * 16, 16)]
                lane = lanes0 + g * 16
                dv = plsc.load_gather(dis_v, [s16])
                for c in range(D):
                    col = plsc.load_gather(rows_v, [lane, full16(c)])
                    plsc.store_scatter(rows_v, [lane, full16(c)], col * dv)
            pltpu.sync_copy(rows_v, acc_s.at[dst_v], add=True)
            return carry
        lax.fori_loop(0, NCHUNK, _chunk, 0)
        plsc.subcore_barrier()
        pltpu.sync_copy(acc_s.at[pl.ds(r0, NPT)], u_out.at[pl.ds(r0, NPT)])


@jax.jit
def kernel(x, edge_index, W_gat, att_src, att_dst, b_gat, W_gcn, b_gcn,
           W_gate, b_gate, gamma, beta):
    n = NPAD
    mask = (jnp.arange(D)[:, None] // C == jnp.arange(H)[None, :]).astype(jnp.float32)
    A_src = mask * att_src.reshape(-1)[:, None]
    A_dst = mask * att_dst.reshape(-1)[:, None]
    R = mask.T
    src = edge_index[0]
    dst = edge_index[1]
    x = jnp.pad(x, ((0, NPAD - N_NODES), (0, 0)))
    z128 = jnp.zeros((n, D), jnp.float32)
    z8 = jnp.zeros((n, 8), jnp.float32)

    blk = 1024
    grid = (n // blk,)
    row_spec = pl.BlockSpec((blk, D), lambda i: (i, 0))
    full = lambda s: pl.BlockSpec(s, lambda i: tuple(0 for _ in s))
    xw, asd, ex_self, init_gat, xg = pl.pallas_call(
        _tc_pre_body,
        grid=grid,
        in_specs=[row_spec, full((D, D)), full((D, H)), full((D, H)),
                  full((H, D)), full((D, D))],
        out_specs=[row_spec, pl.BlockSpec((blk, 2 * H), lambda i: (i, 0)),
                   pl.BlockSpec((blk, H), lambda i: (i, 0)), row_spec, row_spec],
        out_shape=[jax.ShapeDtypeStruct((n, D), jnp.float32),
                   jax.ShapeDtypeStruct((n, 2 * H), jnp.float32),
                   jax.ShapeDtypeStruct((n, H), jnp.float32),
                   jax.ShapeDtypeStruct((n, D), jnp.float32),
                   jax.ShapeDtypeStruct((n, D), jnp.float32)],
    )(x, W_gat, A_src, A_dst, R, W_gcn)

    mesh = plsc.VectorSubcoreMesh(core_axis_name="c", subcore_axis_name="s")
    sc = pl.kernel(
        _sc_body,
        out_type=[jax.ShapeDtypeStruct((n, D), jnp.float32),
                  jax.ShapeDtypeStruct((n, 8), jnp.float32),
                  jax.ShapeDtypeStruct((n, D), jnp.float32),
                  jax.ShapeDtypeStruct((n, 8), jnp.float32)],
        mesh=mesh,
        compiler_params=pltpu.CompilerParams(needs_layout_passes=False, use_tc_tiling_on_sc=False),
        scratch_types=[
            pltpu.VMEM_SHARED((NPAD, D), jnp.float32),
            pltpu.VMEM_SHARED((NPAD, 8), jnp.float32),
            pltpu.VMEM_SHARED((NPAD,), jnp.float32),
            pltpu.VMEM((CHUNK, D), jnp.float32),
            pltpu.VMEM((CHUNK, 8), jnp.float32),
            pltpu.VMEM((CHUNK, 8), jnp.float32),
            pltpu.VMEM((CHUNK, 8), jnp.float32),
            pltpu.VMEM((CHUNK, 8), jnp.float32),
            pltpu.VMEM((NPAD,), jnp.float32),
            pltpu.VMEM((NPT, 8), jnp.float32),
            pltpu.VMEM((NPT,), jnp.float32),
            pltpu.VMEM((CHUNK,), jnp.int32),
            pltpu.VMEM((CHUNK,), jnp.int32),
            pltpu.SemaphoreType.DMA,
        ],
    )
    num, den8, u, deg8 = sc(src, dst, asd, xw, xg, ex_self, init_gat, z128, z8)

    out = pl.pallas_call(
        _tc_post_body,
        grid=grid,
        in_specs=[row_spec, pl.BlockSpec((blk, 8), lambda i: (i, 0)), row_spec,
                  pl.BlockSpec((blk, 8), lambda i: (i, 0)), row_spec, row_spec,
                  full((H, D)), full((D, 2)), full((D, 2)), full((1, 2)),
                  full((1, D)), full((1, D)), full((1, D)), full((1, D))],
        out_specs=row_spec,
        out_shape=jax.ShapeDtypeStruct((n, D), jnp.float32),
    )(num, den8, u, deg8, xg, x, R,
      W_gate[:D], W_gate[D:], b_gate.reshape(1, 2), b_gat.reshape(1, D),
      b_gcn.reshape(1, D), gamma.reshape(1, D), beta.reshape(1, D))
    return out[:N_NODES]


# speedup vs baseline: 12.9849x; 1.1205x over previous
import functools
import jax
import jax.numpy as jnp
from jax import lax
from jax.experimental import pallas as pl
from jax.experimental.pallas import tpu as pltpu
from jax.experimental.pallas import tpu_sc as plsc

N_NODES = 10000
NPAD = 10240
N_EDGES = 320000
D = 128
H = 4
C = 32

NC = 2
NS = 16
EPT = N_EDGES // NS
CHUNK = 80
NCHUNK = EPT // CHUNK
NPT = NPAD // NS


def _tc_pre_body(x_ref, wg_ref, asrc_ref, adst_ref, r_ref, wgcn_ref,
                 xw_ref, asd_ref, exself_ref, initgat_ref, xg_ref):
    x = x_ref[...]
    xw = jnp.dot(x, wg_ref[...], preferred_element_type=jnp.float32)
    a_s = jnp.dot(xw, asrc_ref[...], preferred_element_type=jnp.float32)
    a_d = jnp.dot(xw, adst_ref[...], preferred_element_type=jnp.float32)
    al = a_s + a_d
    al = jnp.where(al > 0, al, 0.2 * al)
    ex = jnp.exp(al)
    xw_ref[...] = xw
    asd_ref[...] = jnp.concatenate([a_s, a_d], axis=1)
    exself_ref[...] = jnp.concatenate([ex, jnp.zeros_like(ex)], axis=1)
    initgat_ref[...] = jnp.dot(ex, r_ref[...], preferred_element_type=jnp.float32) * xw
    xg_ref[...] = jnp.dot(x, wgcn_ref[...], preferred_element_type=jnp.float32)


def _tc_post_body(num_ref, den_ref, u_ref, deg_ref, xg_ref, x_ref, r_ref,
                  w0_ref, w1_ref, bgate_ref, bgat_ref, bgcn_ref, g_ref, b_ref,
                  out_ref):
    den4 = den_ref[:, :4]
    denb = jnp.dot(den4 + 1e-16, r_ref[...], preferred_element_type=jnp.float32)
    gat = num_ref[...] / denb + bgat_ref[...]
    deg = deg_ref[:, 0:1] + 1.0
    dis = lax.rsqrt(deg)
    xg = xg_ref[...]
    gcn = dis * u_ref[...] + (dis * dis) * xg + bgcn_ref[...]
    z0 = (jnp.dot(gat, w0_ref[:, 0:1], preferred_element_type=jnp.float32)
          + jnp.dot(gcn, w1_ref[:, 0:1], preferred_element_type=jnp.float32)
          + bgate_ref[0, 0])
    z1 = (jnp.dot(gat, w0_ref[:, 1:2], preferred_element_type=jnp.float32)
          + jnp.dot(gcn, w1_ref[:, 1:2], preferred_element_type=jnp.float32)
          + bgate_ref[0, 1])
    gw0 = 1.0 / (1.0 + jnp.exp(z1 - z0))
    gw1 = 1.0 - gw0
    y = gw0 * gat + gw1 * gcn + x_ref[...]
    mu = jnp.mean(y, axis=-1, keepdims=True)
    yc = y - mu
    var = jnp.mean(yc * yc, axis=-1, keepdims=True)
    out_ref[...] = g_ref[...] * yc * lax.rsqrt(var + 1e-5) + b_ref[...]


def _edge_pipeline(gat, tile, src_hbm, dst_hbm, tbl_hbm, asd_hbm, acc_s, den_s,
                   rows2, asrc2, adst2, exbuf2, dis_v, src2, dst2, dscat2,
                   isems, gsems, ssems):
    lanes0 = lax.iota(jnp.int32, 16)

    def full16(v):
        return jnp.full((16,), v, jnp.int32)

    def start_idx(c, p):
        off = tile * EPT + c * CHUNK
        pltpu.async_copy(src_hbm.at[pl.ds(off, CHUNK)], src2[p], isems[p])
        pltpu.async_copy(dst_hbm.at[pl.ds(off, CHUNK)], dst2[p], isems[p])

    def wait_idx(p):
        pltpu.make_async_copy(src_hbm.at[pl.ds(0, CHUNK)], src2[p], isems[p]).wait()
        pltpu.make_async_copy(dst_hbm.at[pl.ds(0, CHUNK)], dst2[p], isems[p]).wait()

    def start_gathers(p):
        pltpu.async_copy(tbl_hbm.at[src2[p]], rows2[p], gsems[p])
        if gat:
            pltpu.async_copy(asd_hbm.at[src2[p]], asrc2[p], gsems[p])
            pltpu.async_copy(asd_hbm.at[dst2[p]], adst2[p], gsems[p])

    def wait_gathers(p):
        pltpu.make_async_copy(tbl_hbm.at[src2[p]], rows2[p], gsems[p]).wait()
        if gat:
            pltpu.make_async_copy(asd_hbm.at[src2[p]], asrc2[p], gsems[p]).wait()
            pltpu.make_async_copy(asd_hbm.at[dst2[p]], adst2[p], gsems[p]).wait()

    def start_scatter(p):
        pltpu.async_copy(rows2[p], acc_s.at[dscat2[p]], ssems[p], add=True)
        if gat:
            pltpu.async_copy(exbuf2[p], den_s.at[dscat2[p]], ssems[p], add=True)

    def wait_scatter(p):
        pltpu.make_async_copy(rows2[p], acc_s.at[dscat2[p]], ssems[p]).wait()
        if gat:
            pltpu.make_async_copy(exbuf2[p], den_s.at[dscat2[p]], ssems[p]).wait()

    def compute(p):
        for g in range(CHUNK // 16):
            lane = lanes0 + g * 16
            if gat:
                for h in range(H):
                    av = plsc.load_gather(asrc2[p], [lane, full16(h)])
                    bv = plsc.load_gather(adst2[p], [lane, full16(h + 4)])
                    al = av + bv
                    al = jnp.where(al > 0, al, 0.2 * al)
                    e = jnp.exp(al)
                    plsc.store_scatter(exbuf2[p], [lane, full16(h)], e)
                def _col(j, carry):
                    for u in range(4):
                        c = j * 4 + u
                        sc = plsc.load_gather(exbuf2[p], [lane, full16(c >> 5)])
                        col = plsc.load_gather(rows2[p], [lane, full16(c)])
                        plsc.store_scatter(rows2[p], [lane, full16(c)], col * sc)
                    return carry
                lax.fori_loop(0, D // 4, _col, 0)
            else:
                s16 = src2[p][pl.ds(g * 16, 16)]
                dv = plsc.load_gather(dis_v, [s16])
                def _col(j, carry):
                    for u in range(4):
                        c = j * 4 + u
                        col = plsc.load_gather(rows2[p], [lane, full16(c)])
                        plsc.store_scatter(rows2[p], [lane, full16(c)], col * dv)
                    return carry
                lax.fori_loop(0, D // 4, _col, 0)

    start_idx(0, 0)
    wait_idx(0)
    start_gathers(0)
    start_idx(1, 1)

    def body(i, carry):
        for p in (0, 1):
            ci = 2 * i + p
            if p == 0:
                pl.when(i > 0)(lambda: wait_scatter(1))
            else:
                wait_scatter(0)
            wait_idx(1 - p)
            start_gathers(1 - p)
            wait_gathers(p)
            for g in range(CHUNK // 16):
                dscat2[p][pl.ds(g * 16, 16)] = dst2[p][pl.ds(g * 16, 16)]
            compute(p)
            start_idx(ci + 2, p)
            start_scatter(p)
        return carry
    lax.fori_loop(0, NCHUNK // 2, body, 0)
    wait_gathers(0)
    wait_idx(1)
    wait_scatter(1)


def _sc_body(src_hbm, dst_hbm, asd_hbm, xw_hbm, xg_hbm, exself_hbm,
             initgat_hbm, z128_hbm, z8_hbm,
             num_out, den_out, u_out, deg_out,
             acc_s, den_s, dis_s,
             rows0, rows1, asrc0, asrc1, adst0, adst1, exbuf0, exbuf1,
             ones_v, dis_v, deg_v, dis640_v,
             src0, src1, dst0, dst1, dscat0, dscat1,
             isem0, isem1, gsem0, gsem1, ssem0, ssem1):
    core = lax.axis_index("c")
    tile = lax.axis_index("s")
    r0 = tile * NPT
    lanes0 = lax.iota(jnp.int32, 16)
    rows2, asrc2, adst2 = (rows0, rows1), (asrc0, asrc1), (adst0, adst1)
    exbuf2 = (exbuf0, exbuf1)
    src2, dst2, dscat2 = (src0, src1), (dst0, dst1), (dscat0, dscat1)
    isems, gsems, ssems = (isem0, isem1), (gsem0, gsem1), (ssem0, ssem1)

    def full16(v):
        return jnp.full((16,), v, jnp.int32)

    @pl.when(core == 0)
    def _gat():
        pltpu.sync_copy(initgat_hbm.at[pl.ds(r0, NPT)], acc_s.at[pl.ds(r0, NPT)])
        pltpu.sync_copy(exself_hbm.at[pl.ds(r0, NPT)], den_s.at[pl.ds(r0, NPT)])
        def _zero(k, carry):
            for col in range(4, 8):
                for b in exbuf2:
                    plsc.store_scatter(b, [lanes0 + k * 16, full16(col)],
                                       jnp.zeros((16,), jnp.float32))
            return carry
        lax.fori_loop(0, CHUNK // 16, _zero, 0)
        plsc.subcore_barrier()
        _edge_pipeline(True, tile, src_hbm, dst_hbm, xw_hbm, asd_hbm,
                       acc_s, den_s, rows2, asrc2, adst2, exbuf2, dis_v,
                       src2, dst2, dscat2, isems, gsems, ssems)
        plsc.subcore_barrier()
        pltpu.sync_copy(acc_s.at[pl.ds(r0, NPT)], num_out.at[pl.ds(r0, NPT)])
        pltpu.sync_copy(den_s.at[pl.ds(r0, NPT)], den_out.at[pl.ds(r0, NPT)])

    @pl.when(core == 1)
    def _gcn():
        pltpu.sync_copy(z128_hbm.at[pl.ds(r0, NPT)], acc_s.at[pl.ds(r0, NPT)])
        pltpu.sync_copy(z8_hbm.at[pl.ds(r0, NPT)], den_s.at[pl.ds(r0, NPT)])
        def _ones(k, carry):
            for col in range(8):
                plsc.store_scatter(ones_v, [lanes0 + k * 16, full16(col)],
                                   jnp.ones((16,), jnp.float32))
            return carry
        lax.fori_loop(0, CHUNK // 16, _ones, 0)
        plsc.subcore_barrier()

        def startA(c, p):
            off = tile * EPT + c * CHUNK
            pltpu.async_copy(dst_hbm.at[pl.ds(off, CHUNK)], dst2[p], isems[p])
        def waitAidx(p):
            pltpu.make_async_copy(dst_hbm.at[pl.ds(0, CHUNK)], dst2[p], isems[p]).wait()
        def waitAsc(p):
            pltpu.make_async_copy(ones_v, den_s.at[dscat2[p]], ssems[p]).wait()
        startA(0, 0)
        startA(1, 1)
        def _dbody(i, carry):
            for p in (0, 1):
                ci = 2 * i + p
                waitAidx(p)
                pl.when(i > 0)(lambda: waitAsc(p))
                for g in range(CHUNK // 16):
                    dscat2[p][pl.ds(g * 16, 16)] = dst2[p][pl.ds(g * 16, 16)]
                startA(ci + 2, p)
                pltpu.async_copy(ones_v, den_s.at[dscat2[p]], ssems[p], add=True)
            return carry
        lax.fori_loop(0, NCHUNK // 2, _dbody, 0)
        for p in (0, 1):
            waitAidx(p)
            waitAsc(p)
        plsc.subcore_barrier()
        @pl.when(tile == 0)
        def _wdeg():
            pltpu.sync_copy(den_s, deg_out)

        pltpu.sync_copy(den_s.at[pl.ds(r0, NPT)], deg_v)
        def _newton(j, carry):
            idx = lanes0 + j * 16
            dv = plsc.load_gather(deg_v, [idx, full16(0)])
            xx = dv + 1.0
            ii = plsc.bitcast(xx, jnp.int32)
            ii = jnp.int32(0x5F3759DF) - (ii >> 1)
            y = plsc.bitcast(ii, jnp.float32)
            for _ in range(3):
                y = y * (1.5 - (0.5 * xx) * (y * y))
            plsc.store_scatter(dis640_v, [idx], y)
            return carry
        lax.fori_loop(0, NPT // 16, _newton, 0)
        pltpu.sync_copy(dis640_v, dis_s.at[pl.ds(r0, NPT)])
        plsc.subcore_barrier()
        pltpu.sync_copy(dis_s, dis_v)

        _edge_pipeline(False, tile, src_hbm, dst_hbm, xg_hbm, asd_hbm,
                       acc_s, den_s, rows2, asrc2, adst2, exbuf2, dis_v,
                       src2, dst2, dscat2, isems, gsems, ssems)
        plsc.subcore_barrier()
        pltpu.sync_copy(acc_s.at[pl.ds(r0, NPT)], u_out.at[pl.ds(r0, NPT)])


@jax.jit
def kernel(x, edge_index, W_gat, att_src, att_dst, b_gat, W_gcn, b_gcn,
           W_gate, b_gate, gamma, beta):
    n = NPAD
    mask = (jnp.arange(D)[:, None] // C == jnp.arange(H)[None, :]).astype(jnp.float32)
    A_src = mask * att_src.reshape(-1)[:, None]
    A_dst = mask * att_dst.reshape(-1)[:, None]
    R = mask.T
    pad = jnp.zeros((2 * CHUNK,), jnp.int32)
    src = jnp.concatenate([edge_index[0], pad])
    dst = jnp.concatenate([edge_index[1], pad])
    x = jnp.pad(x, ((0, NPAD - N_NODES), (0, 0)))
    z128 = jnp.zeros((n, D), jnp.float32)
    z8 = jnp.zeros((n, 8), jnp.float32)

    blk = 1024
    grid = (n // blk,)
    row_spec = pl.BlockSpec((blk, D), lambda i: (i, 0))
    full = lambda s: pl.BlockSpec(s, lambda i: tuple(0 for _ in s))
    xw, asd, ex_self, init_gat, xg = pl.pallas_call(
        _tc_pre_body,
        grid=grid,
        in_specs=[row_spec, full((D, D)), full((D, H)), full((D, H)),
                  full((H, D)), full((D, D))],
        out_specs=[row_spec, pl.BlockSpec((blk, 2 * H), lambda i: (i, 0)),
                   pl.BlockSpec((blk, 8), lambda i: (i, 0)), row_spec, row_spec],
        out_shape=[jax.ShapeDtypeStruct((n, D), jnp.float32),
                   jax.ShapeDtypeStruct((n, 2 * H), jnp.float32),
                   jax.ShapeDtypeStruct((n, 8), jnp.float32),
                   jax.ShapeDtypeStruct((n, D), jnp.float32),
                   jax.ShapeDtypeStruct((n, D), jnp.float32)],
    )(x, W_gat, A_src, A_dst, R, W_gcn)

    mesh = plsc.VectorSubcoreMesh(core_axis_name="c", subcore_axis_name="s")
    sc = pl.kernel(
        _sc_body,
        out_type=[jax.ShapeDtypeStruct((n, D), jnp.float32),
                  jax.ShapeDtypeStruct((n, 8), jnp.float32),
                  jax.ShapeDtypeStruct((n, D), jnp.float32),
                  jax.ShapeDtypeStruct((n, 8), jnp.float32)],
        mesh=mesh,
        compiler_params=pltpu.CompilerParams(needs_layout_passes=False, use_tc_tiling_on_sc=False),
        scratch_types=[
            pltpu.VMEM_SHARED((NPAD, D), jnp.float32),
            pltpu.VMEM_SHARED((NPAD, 8), jnp.float32),
            pltpu.VMEM_SHARED((NPAD,), jnp.float32),
            pltpu.VMEM((CHUNK, D), jnp.float32),
            pltpu.VMEM((CHUNK, D), jnp.float32),
            pltpu.VMEM((CHUNK, 8), jnp.float32),
            pltpu.VMEM((CHUNK, 8), jnp.float32),
            pltpu.VMEM((CHUNK, 8), jnp.float32),
            pltpu.VMEM((CHUNK, 8), jnp.float32),
            pltpu.VMEM((CHUNK, 8), jnp.float32),
            pltpu.VMEM((CHUNK, 8), jnp.float32),
            pltpu.VMEM((CHUNK, 8), jnp.float32),
            pltpu.VMEM((NPAD,), jnp.float32),
            pltpu.VMEM((NPT, 8), jnp.float32),
            pltpu.VMEM((NPT,), jnp.float32),
            pltpu.VMEM((CHUNK,), jnp.int32),
            pltpu.VMEM((CHUNK,), jnp.int32),
            pltpu.VMEM((CHUNK,), jnp.int32),
            pltpu.VMEM((CHUNK,), jnp.int32),
            pltpu.VMEM((CHUNK,), jnp.int32),
            pltpu.VMEM((CHUNK,), jnp.int32),
            pltpu.SemaphoreType.DMA,
            pltpu.SemaphoreType.DMA,
            pltpu.SemaphoreType.DMA,
            pltpu.SemaphoreType.DMA,
            pltpu.SemaphoreType.DMA,
            pltpu.SemaphoreType.DMA,
        ],
    )
    num, den8, u, deg8 = sc(src, dst, asd, xw, xg, ex_self, init_gat, z128, z8)

    out = pl.pallas_call(
        _tc_post_body,
        grid=grid,
        in_specs=[row_spec, pl.BlockSpec((blk, 8), lambda i: (i, 0)), row_spec,
                  pl.BlockSpec((blk, 8), lambda i: (i, 0)), row_spec, row_spec,
                  full((H, D)), full((D, 2)), full((D, 2)), full((1, 2)),
                  full((1, D)), full((1, D)), full((1, D)), full((1, D))],
        out_specs=row_spec,
        out_shape=jax.ShapeDtypeStruct((n, D), jnp.float32),
    )(num, den8, u, deg8, xg, x, R,
      W_gate[:D], W_gate[D:], b_gate.reshape(1, 2), b_gat.reshape(1, D),
      b_gcn.reshape(1, D), gamma.reshape(1, D), beta.reshape(1, D))
    return out[:N_NODES]


# speedup vs baseline: 21.5525x; 1.6598x over previous
import functools
import jax
import jax.numpy as jnp
from jax import lax
from jax.experimental import pallas as pl
from jax.experimental.pallas import tpu as pltpu
from jax.experimental.pallas import tpu_sc as plsc

N_NODES = 10000
NPAD = 10240
N_EDGES = 320000
D = 128
H = 4
C = 32

NC = 2
NS = 16
EPT = N_EDGES // NS
CHUNK = 80
NCHUNK = EPT // CHUNK
NPT = NPAD // NS


def _tc_pre_body(x_ref, wg_ref, asrc_ref, adst_ref, r_ref, wgcn_ref,
                 xw_ref, asd_ref, exself_ref, initgat_ref, xg_ref):
    x = x_ref[...]
    xw = jnp.dot(x, wg_ref[...], preferred_element_type=jnp.float32)
    a_s = jnp.dot(xw, asrc_ref[...], preferred_element_type=jnp.float32)
    a_d = jnp.dot(xw, adst_ref[...], preferred_element_type=jnp.float32)
    al = a_s + a_d
    al = jnp.where(al > 0, al, 0.2 * al)
    ex = jnp.exp(al)
    xw_ref[...] = xw
    asd_ref[...] = jnp.concatenate([a_s, a_d], axis=1)
    exself_ref[...] = jnp.concatenate([ex, jnp.zeros_like(ex)], axis=1)
    initgat_ref[...] = jnp.dot(ex, r_ref[...], preferred_element_type=jnp.float32) * xw
    xg_ref[...] = jnp.dot(x, wgcn_ref[...], preferred_element_type=jnp.float32)


def _tc_post_body(num_ref, den_ref, u_ref, deg_ref, xg_ref, x_ref, r_ref,
                  w0_ref, w1_ref, bgate_ref, bgat_ref, bgcn_ref, g_ref, b_ref,
                  out_ref):
    den4 = den_ref[:, :4]
    denb = jnp.dot(den4 + 1e-16, r_ref[...], preferred_element_type=jnp.float32)
    gat = num_ref[...] / denb + bgat_ref[...]
    deg = deg_ref[:, 0:1] + 1.0
    dis = lax.rsqrt(deg)
    xg = xg_ref[...]
    gcn = dis * u_ref[...] + (dis * dis) * xg + bgcn_ref[...]
    z0 = (jnp.dot(gat, w0_ref[:, 0:1], preferred_element_type=jnp.float32)
          + jnp.dot(gcn, w1_ref[:, 0:1], preferred_element_type=jnp.float32)
          + bgate_ref[0, 0])
    z1 = (jnp.dot(gat, w0_ref[:, 1:2], preferred_element_type=jnp.float32)
          + jnp.dot(gcn, w1_ref[:, 1:2], preferred_element_type=jnp.float32)
          + bgate_ref[0, 1])
    gw0 = 1.0 / (1.0 + jnp.exp(z1 - z0))
    gw1 = 1.0 - gw0
    y = gw0 * gat + gw1 * gcn + x_ref[...]
    mu = jnp.mean(y, axis=-1, keepdims=True)
    yc = y - mu
    var = jnp.mean(yc * yc, axis=-1, keepdims=True)
    out_ref[...] = g_ref[...] * yc * lax.rsqrt(var + 1e-5) + b_ref[...]


def _edge_pipeline(gat, tile, src_hbm, dst_hbm, tbl_hbm, asd_hbm, acc_s, den_s,
                   rows2, asrc2, adst2, exbuf2, dis_v, src2, dst2, dscat2,
                   isems, gsems, ssems):
    lanes0 = lax.iota(jnp.int32, 16)

    def full16(v):
        return jnp.full((16,), v, jnp.int32)

    def start_idx(c, p):
        off = tile * EPT + c * CHUNK
        pltpu.async_copy(src_hbm.at[pl.ds(off, CHUNK)], src2[p], isems[p])
        pltpu.async_copy(dst_hbm.at[pl.ds(off, CHUNK)], dst2[p], isems[p])

    def wait_idx(p):
        pltpu.make_async_copy(src_hbm.at[pl.ds(0, CHUNK)], src2[p], isems[p]).wait()
        pltpu.make_async_copy(dst_hbm.at[pl.ds(0, CHUNK)], dst2[p], isems[p]).wait()

    def start_gathers(p):
        pltpu.async_copy(tbl_hbm.at[src2[p]], rows2[p], gsems[p])
        if gat:
            pltpu.async_copy(asd_hbm.at[src2[p]], asrc2[p], gsems[p])
            pltpu.async_copy(asd_hbm.at[dst2[p]], adst2[p], gsems[p])

    def wait_gathers(p):
        pltpu.make_async_copy(tbl_hbm.at[src2[p]], rows2[p], gsems[p]).wait()
        if gat:
            pltpu.make_async_copy(asd_hbm.at[src2[p]], asrc2[p], gsems[p]).wait()
            pltpu.make_async_copy(asd_hbm.at[dst2[p]], adst2[p], gsems[p]).wait()

    def start_scatter(p):
        pltpu.async_copy(rows2[p], acc_s.at[dscat2[p]], ssems[p], add=True)
        if gat:
            pltpu.async_copy(exbuf2[p], den_s.at[dscat2[p]], ssems[p], add=True)

    def wait_scatter(p):
        pltpu.make_async_copy(rows2[p], acc_s.at[dscat2[p]], ssems[p]).wait()
        if gat:
            pltpu.make_async_copy(exbuf2[p], den_s.at[dscat2[p]], ssems[p]).wait()

    def compute(p):
        for g in range(CHUNK // 16):
            lane = lanes0 + g * 16
            if gat:
                for h in range(H):
                    av = plsc.load_gather(asrc2[p], [lane, full16(h)])
                    bv = plsc.load_gather(adst2[p], [lane, full16(h + 4)])
                    al = av + bv
                    al = jnp.where(al > 0, al, 0.2 * al)
                    e = jnp.exp(al)
                    plsc.store_scatter(exbuf2[p], [lane, full16(h)], e)
                @plsc.parallel_loop(0, D, step=4)
                def _col(c0):
                    for u in range(4):
                        c = c0 + u
                        sc = plsc.load_gather(exbuf2[p], [lane, full16(c >> 5)])
                        col = plsc.load_gather(rows2[p], [lane, full16(c)])
                        plsc.store_scatter(rows2[p], [lane, full16(c)], col * sc)
            else:
                s16 = src2[p][pl.ds(g * 16, 16)]
                dv = plsc.load_gather(dis_v, [s16])
                @plsc.parallel_loop(0, D, step=4)
                def _col(c0):
                    for u in range(4):
                        c = c0 + u
                        col = plsc.load_gather(rows2[p], [lane, full16(c)])
                        plsc.store_scatter(rows2[p], [lane, full16(c)], col * dv)

    start_idx(0, 0)
    wait_idx(0)
    start_gathers(0)
    start_idx(1, 1)

    def body(i, carry):
        for p in (0, 1):
            ci = 2 * i + p
            if p == 0:
                pl.when(i > 0)(lambda: wait_scatter(1))
            else:
                wait_scatter(0)
            wait_idx(1 - p)
            start_gathers(1 - p)
            wait_gathers(p)
            for g in range(CHUNK // 16):
                dscat2[p][pl.ds(g * 16, 16)] = dst2[p][pl.ds(g * 16, 16)]
            compute(p)
            start_idx(ci + 2, p)
            start_scatter(p)
        return carry
    lax.fori_loop(0, NCHUNK // 2, body, 0)
    wait_gathers(0)
    wait_idx(1)
    wait_scatter(1)


def _sc_body(src_hbm, dst_hbm, asd_hbm, xw_hbm, xg_hbm, exself_hbm,
             initgat_hbm, z128_hbm, z8_hbm,
             num_out, den_out, u_out, deg_out,
             acc_s, den_s, dis_s,
             rows0, rows1, asrc0, asrc1, adst0, adst1, exbuf0, exbuf1,
             ones_v, dis_v, deg_v, dis640_v,
             src0, src1, dst0, dst1, dscat0, dscat1,
             isem0, isem1, gsem0, gsem1, ssem0, ssem1):
    core = lax.axis_index("c")
    tile = lax.axis_index("s")
    r0 = tile * NPT
    lanes0 = lax.iota(jnp.int32, 16)
    rows2, asrc2, adst2 = (rows0, rows1), (asrc0, asrc1), (adst0, adst1)
    exbuf2 = (exbuf0, exbuf1)
    src2, dst2, dscat2 = (src0, src1), (dst0, dst1), (dscat0, dscat1)
    isems, gsems, ssems = (isem0, isem1), (gsem0, gsem1), (ssem0, ssem1)

    def full16(v):
        return jnp.full((16,), v, jnp.int32)

    @pl.when(core == 0)
    def _gat():
        pltpu.sync_copy(initgat_hbm.at[pl.ds(r0, NPT)], acc_s.at[pl.ds(r0, NPT)])
        pltpu.sync_copy(exself_hbm.at[pl.ds(r0, NPT)], den_s.at[pl.ds(r0, NPT)])
        def _zero(k, carry):
            for col in range(4, 8):
                for b in exbuf2:
                    plsc.store_scatter(b, [lanes0 + k * 16, full16(col)],
                                       jnp.zeros((16,), jnp.float32))
            return carry
        lax.fori_loop(0, CHUNK // 16, _zero, 0)
        plsc.subcore_barrier()
        _edge_pipeline(True, tile, src_hbm, dst_hbm, xw_hbm, asd_hbm,
                       acc_s, den_s, rows2, asrc2, adst2, exbuf2, dis_v,
                       src2, dst2, dscat2, isems, gsems, ssems)
        plsc.subcore_barrier()
        pltpu.sync_copy(acc_s.at[pl.ds(r0, NPT)], num_out.at[pl.ds(r0, NPT)])
        pltpu.sync_copy(den_s.at[pl.ds(r0, NPT)], den_out.at[pl.ds(r0, NPT)])

    @pl.when(core == 1)
    def _gcn():
        pltpu.sync_copy(z128_hbm.at[pl.ds(r0, NPT)], acc_s.at[pl.ds(r0, NPT)])
        pltpu.sync_copy(z8_hbm.at[pl.ds(r0, NPT)], den_s.at[pl.ds(r0, NPT)])
        def _ones(k, carry):
            for col in range(8):
                plsc.store_scatter(ones_v, [lanes0 + k * 16, full16(col)],
                                   jnp.ones((16,), jnp.float32))
            return carry
        lax.fori_loop(0, CHUNK // 16, _ones, 0)
        plsc.subcore_barrier()

        def startA(c, p):
            off = tile * EPT + c * CHUNK
            pltpu.async_copy(dst_hbm.at[pl.ds(off, CHUNK)], dst2[p], isems[p])
        def waitAidx(p):
            pltpu.make_async_copy(dst_hbm.at[pl.ds(0, CHUNK)], dst2[p], isems[p]).wait()
        def waitAsc(p):
            pltpu.make_async_copy(ones_v, den_s.at[dscat2[p]], ssems[p]).wait()
        startA(0, 0)
        startA(1, 1)
        def _dbody(i, carry):
            for p in (0, 1):
                ci = 2 * i + p
                waitAidx(p)
                pl.when(i > 0)(lambda: waitAsc(p))
                for g in range(CHUNK // 16):
                    dscat2[p][pl.ds(g * 16, 16)] = dst2[p][pl.ds(g * 16, 16)]
                startA(ci + 2, p)
                pltpu.async_copy(ones_v, den_s.at[dscat2[p]], ssems[p], add=True)
            return carry
        lax.fori_loop(0, NCHUNK // 2, _dbody, 0)
        for p in (0, 1):
            waitAidx(p)
            waitAsc(p)
        plsc.subcore_barrier()
        @pl.when(tile == 0)
        def _wdeg():
            pltpu.sync_copy(den_s, deg_out)

        pltpu.sync_copy(den_s.at[pl.ds(r0, NPT)], deg_v)
        def _newton(j, carry):
            idx = lanes0 + j * 16
            dv = plsc.load_gather(deg_v, [idx, full16(0)])
            xx = dv + 1.0
            ii = plsc.bitcast(xx, jnp.int32)
            ii = jnp.int32(0x5F3759DF) - (ii >> 1)
            y = plsc.bitcast(ii, jnp.float32)
            for _ in range(3):
                y = y * (1.5 - (0.5 * xx) * (y * y))
            plsc.store_scatter(dis640_v, [idx], y)
            return carry
        lax.fori_loop(0, NPT // 16, _newton, 0)
        pltpu.sync_copy(dis640_v, dis_s.at[pl.ds(r0, NPT)])
        plsc.subcore_barrier()
        pltpu.sync_copy(dis_s, dis_v)

        _edge_pipeline(False, tile, src_hbm, dst_hbm, xg_hbm, asd_hbm,
                       acc_s, den_s, rows2, asrc2, adst2, exbuf2, dis_v,
                       src2, dst2, dscat2, isems, gsems, ssems)
        plsc.subcore_barrier()
        pltpu.sync_copy(acc_s.at[pl.ds(r0, NPT)], u_out.at[pl.ds(r0, NPT)])


@jax.jit
def kernel(x, edge_index, W_gat, att_src, att_dst, b_gat, W_gcn, b_gcn,
           W_gate, b_gate, gamma, beta):
    n = NPAD
    mask = (jnp.arange(D)[:, None] // C == jnp.arange(H)[None, :]).astype(jnp.float32)
    A_src = mask * att_src.reshape(-1)[:, None]
    A_dst = mask * att_dst.reshape(-1)[:, None]
    R = mask.T
    pad = jnp.zeros((2 * CHUNK,), jnp.int32)
    src = jnp.concatenate([edge_index[0], pad])
    dst = jnp.concatenate([edge_index[1], pad])
    x = jnp.pad(x, ((0, NPAD - N_NODES), (0, 0)))
    z128 = jnp.zeros((n, D), jnp.float32)
    z8 = jnp.zeros((n, 8), jnp.float32)

    blk = 1024
    grid = (n // blk,)
    row_spec = pl.BlockSpec((blk, D), lambda i: (i, 0))
    full = lambda s: pl.BlockSpec(s, lambda i: tuple(0 for _ in s))
    xw, asd, ex_self, init_gat, xg = pl.pallas_call(
        _tc_pre_body,
        grid=grid,
        in_specs=[row_spec, full((D, D)), full((D, H)), full((D, H)),
                  full((H, D)), full((D, D))],
        out_specs=[row_spec, pl.BlockSpec((blk, 2 * H), lambda i: (i, 0)),
                   pl.BlockSpec((blk, 8), lambda i: (i, 0)), row_spec, row_spec],
        out_shape=[jax.ShapeDtypeStruct((n, D), jnp.float32),
                   jax.ShapeDtypeStruct((n, 2 * H), jnp.float32),
                   jax.ShapeDtypeStruct((n, 8), jnp.float32),
                   jax.ShapeDtypeStruct((n, D), jnp.float32),
                   jax.ShapeDtypeStruct((n, D), jnp.float32)],
    )(x, W_gat, A_src, A_dst, R, W_gcn)

    mesh = plsc.VectorSubcoreMesh(core_axis_name="c", subcore_axis_name="s")
    sc = pl.kernel(
        _sc_body,
        out_type=[jax.ShapeDtypeStruct((n, D), jnp.float32),
                  jax.ShapeDtypeStruct((n, 8), jnp.float32),
                  jax.ShapeDtypeStruct((n, D), jnp.float32),
                  jax.ShapeDtypeStruct((n, 8), jnp.float32)],
        mesh=mesh,
        compiler_params=pltpu.CompilerParams(needs_layout_passes=False, use_tc_tiling_on_sc=False),
        scratch_types=[
            pltpu.VMEM_SHARED((NPAD, D), jnp.float32),
            pltpu.VMEM_SHARED((NPAD, 8), jnp.float32),
            pltpu.VMEM_SHARED((NPAD,), jnp.float32),
            pltpu.VMEM((CHUNK, D), jnp.float32),
            pltpu.VMEM((CHUNK, D), jnp.float32),
            pltpu.VMEM((CHUNK, 8), jnp.float32),
            pltpu.VMEM((CHUNK, 8), jnp.float32),
            pltpu.VMEM((CHUNK, 8), jnp.float32),
            pltpu.VMEM((CHUNK, 8), jnp.float32),
            pltpu.VMEM((CHUNK, 8), jnp.float32),
            pltpu.VMEM((CHUNK, 8), jnp.float32),
            pltpu.VMEM((CHUNK, 8), jnp.float32),
            pltpu.VMEM((NPAD,), jnp.float32),
            pltpu.VMEM((NPT, 8), jnp.float32),
            pltpu.VMEM((NPT,), jnp.float32),
            pltpu.VMEM((CHUNK,), jnp.int32),
            pltpu.VMEM((CHUNK,), jnp.int32),
            pltpu.VMEM((CHUNK,), jnp.int32),
            pltpu.VMEM((CHUNK,), jnp.int32),
            pltpu.VMEM((CHUNK,), jnp.int32),
            pltpu.VMEM((CHUNK,), jnp.int32),
            pltpu.SemaphoreType.DMA,
            pltpu.SemaphoreType.DMA,
            pltpu.SemaphoreType.DMA,
            pltpu.SemaphoreType.DMA,
            pltpu.SemaphoreType.DMA,
            pltpu.SemaphoreType.DMA,
        ],
    )
    num, den8, u, deg8 = sc(src, dst, asd, xw, xg, ex_self, init_gat, z128, z8)

    out = pl.pallas_call(
        _tc_post_body,
        grid=grid,
        in_specs=[row_spec, pl.BlockSpec((blk, 8), lambda i: (i, 0)), row_spec,
                  pl.BlockSpec((blk, 8), lambda i: (i, 0)), row_spec, row_spec,
                  full((H, D)), full((D, 2)), full((D, 2)), full((1, 2)),
                  full((1, D)), full((1, D)), full((1, D)), full((1, D))],
        out_specs=row_spec,
        out_shape=jax.ShapeDtypeStruct((n, D), jnp.float32),
    )(num, den8, u, deg8, xg, x, R,
      W_gate[:D], W_gate[D:], b_gate.reshape(1, 2), b_gat.reshape(1, D),
      b_gcn.reshape(1, D), gamma.reshape(1, D), beta.reshape(1, D))
    return out[:N_NODES]


# speedup vs baseline: 21.6264x; 1.0034x over previous
import functools
import jax
import jax.numpy as jnp
from jax import lax
from jax.experimental import pallas as pl
from jax.experimental.pallas import tpu as pltpu
from jax.experimental.pallas import tpu_sc as plsc

N_NODES = 10000
NPAD = 10240
N_EDGES = 320000
D = 128
H = 4
C = 32

NC = 2
NS = 16
EPT = N_EDGES // NS
CHUNK = 80
NCHUNK = EPT // CHUNK
NPT = NPAD // NS


def _tc_pre_body(x_ref, wg_ref, asrc_ref, adst_ref, r_ref, wgcn_ref,
                 xw_ref, asd_ref, exself_ref, initgat_ref, xg_ref):
    x = x_ref[...]
    xw = jnp.dot(x, wg_ref[...], preferred_element_type=jnp.float32)
    a_s = jnp.dot(xw, asrc_ref[...], preferred_element_type=jnp.float32)
    a_d = jnp.dot(xw, adst_ref[...], preferred_element_type=jnp.float32)
    al = a_s + a_d
    al = jnp.where(al > 0, al, 0.2 * al)
    ex = jnp.exp(al)
    xw_ref[...] = xw
    asd_ref[...] = jnp.concatenate([a_s, a_d], axis=1)
    exself_ref[...] = jnp.concatenate([ex, jnp.zeros_like(ex)], axis=1)
    initgat_ref[...] = jnp.dot(ex, r_ref[...], preferred_element_type=jnp.float32) * xw
    xg_ref[...] = jnp.dot(x, wgcn_ref[...], preferred_element_type=jnp.float32)


def _tc_post_body(num_ref, den_ref, u_ref, deg_ref, xg_ref, x_ref, r_ref,
                  w0_ref, w1_ref, bgate_ref, bgat_ref, bgcn_ref, g_ref, b_ref,
                  out_ref):
    den4 = den_ref[:, :4]
    denb = jnp.dot(den4 + 1e-16, r_ref[...], preferred_element_type=jnp.float32)
    gat = num_ref[...] / denb + bgat_ref[...]
    deg = deg_ref[:, 0:1] + 1.0
    dis = lax.rsqrt(deg)
    xg = xg_ref[...]
    gcn = dis * u_ref[...] + (dis * dis) * xg + bgcn_ref[...]
    z0 = (jnp.dot(gat, w0_ref[:, 0:1], preferred_element_type=jnp.float32)
          + jnp.dot(gcn, w1_ref[:, 0:1], preferred_element_type=jnp.float32)
          + bgate_ref[0, 0])
    z1 = (jnp.dot(gat, w0_ref[:, 1:2], preferred_element_type=jnp.float32)
          + jnp.dot(gcn, w1_ref[:, 1:2], preferred_element_type=jnp.float32)
          + bgate_ref[0, 1])
    gw0 = 1.0 / (1.0 + jnp.exp(z1 - z0))
    gw1 = 1.0 - gw0
    y = gw0 * gat + gw1 * gcn + x_ref[...]
    mu = jnp.mean(y, axis=-1, keepdims=True)
    yc = y - mu
    var = jnp.mean(yc * yc, axis=-1, keepdims=True)
    out_ref[...] = g_ref[...] * yc * lax.rsqrt(var + 1e-5) + b_ref[...]


def _edge_pipeline(gat, tile, src_hbm, dst_hbm, tbl_hbm, asd_hbm, acc_s, den_s,
                   rows2, asrc2, adst2, exbuf2, dis_v, src2, dst2, dscat2,
                   isems, gsems, ssems):
    lanes0 = lax.iota(jnp.int32, 16)

    def full16(v):
        return jnp.full((16,), v, jnp.int32)

    def start_idx(c, p):
        off = tile * EPT + c * CHUNK
        pltpu.async_copy(src_hbm.at[pl.ds(off, CHUNK)], src2[p], isems[p])
        pltpu.async_copy(dst_hbm.at[pl.ds(off, CHUNK)], dst2[p], isems[p])

    def wait_idx(p):
        pltpu.make_async_copy(src_hbm.at[pl.ds(0, CHUNK)], src2[p], isems[p]).wait()
        pltpu.make_async_copy(dst_hbm.at[pl.ds(0, CHUNK)], dst2[p], isems[p]).wait()

    def start_gathers(p):
        pltpu.async_copy(tbl_hbm.at[src2[p]], rows2[p], gsems[p])
        if gat:
            pltpu.async_copy(asd_hbm.at[src2[p]], asrc2[p], gsems[p])
            pltpu.async_copy(asd_hbm.at[dst2[p]], adst2[p], gsems[p])

    def wait_gathers(p):
        pltpu.make_async_copy(tbl_hbm.at[src2[p]], rows2[p], gsems[p]).wait()
        if gat:
            pltpu.make_async_copy(asd_hbm.at[src2[p]], asrc2[p], gsems[p]).wait()
            pltpu.make_async_copy(asd_hbm.at[dst2[p]], adst2[p], gsems[p]).wait()

    def start_scatter(p):
        pltpu.async_copy(rows2[p], acc_s.at[dscat2[p]], ssems[p], add=True)
        if gat:
            pltpu.async_copy(exbuf2[p], den_s.at[dscat2[p]], ssems[p], add=True)

    def wait_scatter(p):
        pltpu.make_async_copy(rows2[p], acc_s.at[dscat2[p]], ssems[p]).wait()
        if gat:
            pltpu.make_async_copy(exbuf2[p], den_s.at[dscat2[p]], ssems[p]).wait()

    def compute(p):
        for g in range(CHUNK // 16):
            lane = lanes0 + g * 16
            if gat:
                for h in range(H):
                    av = plsc.load_gather(asrc2[p], [lane, full16(h)])
                    bv = plsc.load_gather(adst2[p], [lane, full16(h + 4)])
                    al = av + bv
                    al = jnp.where(al > 0, al, 0.2 * al)
                    e = jnp.exp(al)
                    plsc.store_scatter(exbuf2[p], [lane, full16(h)], e)
                    @plsc.parallel_loop(h * C, (h + 1) * C, step=4)
                    def _col(c0):
                        for u in range(4):
                            c = c0 + u
                            col = plsc.load_gather(rows2[p], [lane, full16(c)])
                            plsc.store_scatter(rows2[p], [lane, full16(c)], col * e)
            else:
                s16 = src2[p][pl.ds(g * 16, 16)]
                dv = plsc.load_gather(dis_v, [s16])
                @plsc.parallel_loop(0, D, step=4)
                def _col(c0):
                    for u in range(4):
                        c = c0 + u
                        col = plsc.load_gather(rows2[p], [lane, full16(c)])
                        plsc.store_scatter(rows2[p], [lane, full16(c)], col * dv)

    start_idx(0, 0)
    wait_idx(0)
    start_gathers(0)
    start_idx(1, 1)

    def body(i, carry):
        for p in (0, 1):
            ci = 2 * i + p
            if p == 0:
                pl.when(i > 0)(lambda: wait_scatter(1))
            else:
                wait_scatter(0)
            wait_idx(1 - p)
            start_gathers(1 - p)
            wait_gathers(p)
            for g in range(CHUNK // 16):
                dscat2[p][pl.ds(g * 16, 16)] = dst2[p][pl.ds(g * 16, 16)]
            compute(p)
            start_idx(ci + 2, p)
            start_scatter(p)
        return carry
    lax.fori_loop(0, NCHUNK // 2, body, 0)
    wait_gathers(0)
    wait_idx(1)
    wait_scatter(1)


def _sc_body(src_hbm, dst_hbm, asd_hbm, xw_hbm, xg_hbm, exself_hbm,
             initgat_hbm, z128_hbm, z8_hbm,
             num_out, den_out, u_out, deg_out,
             acc_s, den_s, dis_s,
             rows0, rows1, asrc0, asrc1, adst0, adst1, exbuf0, exbuf1,
             ones_v, dis_v, deg_v, dis640_v,
             src0, src1, dst0, dst1, dscat0, dscat1,
             isem0, isem1, gsem0, gsem1, ssem0, ssem1):
    core = lax.axis_index("c")
    tile = lax.axis_index("s")
    r0 = tile * NPT
    lanes0 = lax.iota(jnp.int32, 16)
    rows2, asrc2, adst2 = (rows0, rows1), (asrc0, asrc1), (adst0, adst1)
    exbuf2 = (exbuf0, exbuf1)
    src2, dst2, dscat2 = (src0, src1), (dst0, dst1), (dscat0, dscat1)
    isems, gsems, ssems = (isem0, isem1), (gsem0, gsem1), (ssem0, ssem1)

    def full16(v):
        return jnp.full((16,), v, jnp.int32)

    @pl.when(core == 0)
    def _gat():
        pltpu.sync_copy(initgat_hbm.at[pl.ds(r0, NPT)], acc_s.at[pl.ds(r0, NPT)])
        pltpu.sync_copy(exself_hbm.at[pl.ds(r0, NPT)], den_s.at[pl.ds(r0, NPT)])
        def _zero(k, carry):
            for col in range(4, 8):
                for b in exbuf2:
                    plsc.store_scatter(b, [lanes0 + k * 16, full16(col)],
                                       jnp.zeros((16,), jnp.float32))
            return carry
        lax.fori_loop(0, CHUNK // 16, _zero, 0)
        plsc.subcore_barrier()
        _edge_pipeline(True, tile, src_hbm, dst_hbm, xw_hbm, asd_hbm,
                       acc_s, den_s, rows2, asrc2, adst2, exbuf2, dis_v,
                       src2, dst2, dscat2, isems, gsems, ssems)
        plsc.subcore_barrier()
        pltpu.sync_copy(acc_s.at[pl.ds(r0, NPT)], num_out.at[pl.ds(r0, NPT)])
        pltpu.sync_copy(den_s.at[pl.ds(r0, NPT)], den_out.at[pl.ds(r0, NPT)])

    @pl.when(core == 1)
    def _gcn():
        pltpu.sync_copy(z128_hbm.at[pl.ds(r0, NPT)], acc_s.at[pl.ds(r0, NPT)])
        pltpu.sync_copy(z8_hbm.at[pl.ds(r0, NPT)], den_s.at[pl.ds(r0, NPT)])
        def _ones(k, carry):
            for col in range(8):
                plsc.store_scatter(ones_v, [lanes0 + k * 16, full16(col)],
                                   jnp.ones((16,), jnp.float32))
            return carry
        lax.fori_loop(0, CHUNK // 16, _ones, 0)
        plsc.subcore_barrier()

        def startA(c, p):
            off = tile * EPT + c * CHUNK
            pltpu.async_copy(dst_hbm.at[pl.ds(off, CHUNK)], dst2[p], isems[p])
        def waitAidx(p):
            pltpu.make_async_copy(dst_hbm.at[pl.ds(0, CHUNK)], dst2[p], isems[p]).wait()
        def waitAsc(p):
            pltpu.make_async_copy(ones_v, den_s.at[dscat2[p]], ssems[p]).wait()
        startA(0, 0)
        startA(1, 1)
        def _dbody(i, carry):
            for p in (0, 1):
                ci = 2 * i + p
                waitAidx(p)
                pl.when(i > 0)(lambda: waitAsc(p))
                for g in range(CHUNK // 16):
                    dscat2[p][pl.ds(g * 16, 16)] = dst2[p][pl.ds(g * 16, 16)]
                startA(ci + 2, p)
                pltpu.async_copy(ones_v, den_s.at[dscat2[p]], ssems[p], add=True)
            return carry
        lax.fori_loop(0, NCHUNK // 2, _dbody, 0)
        for p in (0, 1):
            waitAidx(p)
            waitAsc(p)
        plsc.subcore_barrier()
        @pl.when(tile == 0)
        def _wdeg():
            pltpu.sync_copy(den_s, deg_out)

        pltpu.sync_copy(den_s.at[pl.ds(r0, NPT)], deg_v)
        def _newton(j, carry):
            idx = lanes0 + j * 16
            dv = plsc.load_gather(deg_v, [idx, full16(0)])
            xx = dv + 1.0
            ii = plsc.bitcast(xx, jnp.int32)
            ii = jnp.int32(0x5F3759DF) - (ii >> 1)
            y = plsc.bitcast(ii, jnp.float32)
            for _ in range(3):
                y = y * (1.5 - (0.5 * xx) * (y * y))
            plsc.store_scatter(dis640_v, [idx], y)
            return carry
        lax.fori_loop(0, NPT // 16, _newton, 0)
        pltpu.sync_copy(dis640_v, dis_s.at[pl.ds(r0, NPT)])
        plsc.subcore_barrier()
        pltpu.sync_copy(dis_s, dis_v)

        _edge_pipeline(False, tile, src_hbm, dst_hbm, xg_hbm, asd_hbm,
                       acc_s, den_s, rows2, asrc2, adst2, exbuf2, dis_v,
                       src2, dst2, dscat2, isems, gsems, ssems)
        plsc.subcore_barrier()
        pltpu.sync_copy(acc_s.at[pl.ds(r0, NPT)], u_out.at[pl.ds(r0, NPT)])


@jax.jit
def kernel(x, edge_index, W_gat, att_src, att_dst, b_gat, W_gcn, b_gcn,
           W_gate, b_gate, gamma, beta):
    n = NPAD
    mask = (jnp.arange(D)[:, None] // C == jnp.arange(H)[None, :]).astype(jnp.float32)
    A_src = mask * att_src.reshape(-1)[:, None]
    A_dst = mask * att_dst.reshape(-1)[:, None]
    R = mask.T
    pad = jnp.zeros((2 * CHUNK,), jnp.int32)
    src = jnp.concatenate([edge_index[0], pad])
    dst = jnp.concatenate([edge_index[1], pad])
    x = jnp.pad(x, ((0, NPAD - N_NODES), (0, 0)))
    z128 = jnp.zeros((n, D), jnp.float32)
    z8 = jnp.zeros((n, 8), jnp.float32)

    blk = 1024
    grid = (n // blk,)
    row_spec = pl.BlockSpec((blk, D), lambda i: (i, 0))
    full = lambda s: pl.BlockSpec(s, lambda i: tuple(0 for _ in s))
    xw, asd, ex_self, init_gat, xg = pl.pallas_call(
        _tc_pre_body,
        grid=grid,
        in_specs=[row_spec, full((D, D)), full((D, H)), full((D, H)),
                  full((H, D)), full((D, D))],
        out_specs=[row_spec, pl.BlockSpec((blk, 2 * H), lambda i: (i, 0)),
                   pl.BlockSpec((blk, 8), lambda i: (i, 0)), row_spec, row_spec],
        out_shape=[jax.ShapeDtypeStruct((n, D), jnp.float32),
                   jax.ShapeDtypeStruct((n, 2 * H), jnp.float32),
                   jax.ShapeDtypeStruct((n, 8), jnp.float32),
                   jax.ShapeDtypeStruct((n, D), jnp.float32),
                   jax.ShapeDtypeStruct((n, D), jnp.float32)],
    )(x, W_gat, A_src, A_dst, R, W_gcn)

    mesh = plsc.VectorSubcoreMesh(core_axis_name="c", subcore_axis_name="s")
    sc = pl.kernel(
        _sc_body,
        out_type=[jax.ShapeDtypeStruct((n, D), jnp.float32),
                  jax.ShapeDtypeStruct((n, 8), jnp.float32),
                  jax.ShapeDtypeStruct((n, D), jnp.float32),
                  jax.ShapeDtypeStruct((n, 8), jnp.float32)],
        mesh=mesh,
        compiler_params=pltpu.CompilerParams(needs_layout_passes=False, use_tc_tiling_on_sc=False),
        scratch_types=[
            pltpu.VMEM_SHARED((NPAD, D), jnp.float32),
            pltpu.VMEM_SHARED((NPAD, 8), jnp.float32),
            pltpu.VMEM_SHARED((NPAD,), jnp.float32),
            pltpu.VMEM((CHUNK, D), jnp.float32),
            pltpu.VMEM((CHUNK, D), jnp.float32),
            pltpu.VMEM((CHUNK, 8), jnp.float32),
            pltpu.VMEM((CHUNK, 8), jnp.float32),
            pltpu.VMEM((CHUNK, 8), jnp.float32),
            pltpu.VMEM((CHUNK, 8), jnp.float32),
            pltpu.VMEM((CHUNK, 8), jnp.float32),
            pltpu.VMEM((CHUNK, 8), jnp.float32),
            pltpu.VMEM((CHUNK, 8), jnp.float32),
            pltpu.VMEM((NPAD,), jnp.float32),
            pltpu.VMEM((NPT, 8), jnp.float32),
            pltpu.VMEM((NPT,), jnp.float32),
            pltpu.VMEM((CHUNK,), jnp.int32),
            pltpu.VMEM((CHUNK,), jnp.int32),
            pltpu.VMEM((CHUNK,), jnp.int32),
            pltpu.VMEM((CHUNK,), jnp.int32),
            pltpu.VMEM((CHUNK,), jnp.int32),
            pltpu.VMEM((CHUNK,), jnp.int32),
            pltpu.SemaphoreType.DMA,
            pltpu.SemaphoreType.DMA,
            pltpu.SemaphoreType.DMA,
            pltpu.SemaphoreType.DMA,
            pltpu.SemaphoreType.DMA,
            pltpu.SemaphoreType.DMA,
        ],
    )
    num, den8, u, deg8 = sc(src, dst, asd, xw, xg, ex_self, init_gat, z128, z8)

    out = pl.pallas_call(
        _tc_post_body,
        grid=grid,
        in_specs=[row_spec, pl.BlockSpec((blk, 8), lambda i: (i, 0)), row_spec,
                  pl.BlockSpec((blk, 8), lambda i: (i, 0)), row_spec, row_spec,
                  full((H, D)), full((D, 2)), full((D, 2)), full((1, 2)),
                  full((1, D)), full((1, D)), full((1, D)), full((1, D))],
        out_specs=row_spec,
        out_shape=jax.ShapeDtypeStruct((n, D), jnp.float32),
    )(num, den8, u, deg8, xg, x, R,
      W_gate[:D], W_gate[D:], b_gate.reshape(1, 2), b_gat.reshape(1, D),
      b_gcn.reshape(1, D), gamma.reshape(1, D), beta.reshape(1, D))
    return out[:N_NODES]


# speedup vs baseline: 23.3659x; 1.0804x over previous
import functools
import jax
import jax.numpy as jnp
from jax import lax
from jax.experimental import pallas as pl
from jax.experimental.pallas import tpu as pltpu
from jax.experimental.pallas import tpu_sc as plsc

N_NODES = 10000
NPAD = 10240
N_EDGES = 320000
D = 128
H = 4
C = 32

NC = 2
NS = 16
EPT = N_EDGES // NS
CHUNK = 80
NCHUNK = EPT // CHUNK
NPT = NPAD // NS


def _tc_pre_body(x_ref, wg_ref, asrc_ref, adst_ref, r_ref, wgcn_ref,
                 xw_ref, asd_ref, exself_ref, initgat_ref, xg_ref):
    x = x_ref[...]
    xw = jnp.dot(x, wg_ref[...], preferred_element_type=jnp.float32)
    a_s = jnp.dot(xw, asrc_ref[...], preferred_element_type=jnp.float32)
    a_d = jnp.dot(xw, adst_ref[...], preferred_element_type=jnp.float32)
    al = a_s + a_d
    al = jnp.where(al > 0, al, 0.2 * al)
    ex = jnp.exp(al)
    xw_ref[...] = xw
    asd_ref[...] = jnp.concatenate([a_s, a_d], axis=1)
    exself_ref[...] = jnp.concatenate([ex, jnp.zeros_like(ex)], axis=1)
    initgat_ref[...] = jnp.dot(ex, r_ref[...], preferred_element_type=jnp.float32) * xw
    xg_ref[...] = jnp.dot(x, wgcn_ref[...], preferred_element_type=jnp.float32)


def _tc_post_body(num_ref, den_ref, u_ref, deg_ref, xg_ref, x_ref, r_ref,
                  w0_ref, w1_ref, bgate_ref, bgat_ref, bgcn_ref, g_ref, b_ref,
                  out_ref):
    den4 = den_ref[:, :4]
    denb = jnp.dot(den4 + 1e-16, r_ref[...], preferred_element_type=jnp.float32)
    gat = num_ref[...] / denb + bgat_ref[...]
    deg = deg_ref[:, 0:1] + 1.0
    dis = lax.rsqrt(deg)
    xg = xg_ref[...]
    gcn = dis * u_ref[...] + (dis * dis) * xg + bgcn_ref[...]
    z0 = (jnp.dot(gat, w0_ref[:, 0:1], preferred_element_type=jnp.float32)
          + jnp.dot(gcn, w1_ref[:, 0:1], preferred_element_type=jnp.float32)
          + bgate_ref[0, 0])
    z1 = (jnp.dot(gat, w0_ref[:, 1:2], preferred_element_type=jnp.float32)
          + jnp.dot(gcn, w1_ref[:, 1:2], preferred_element_type=jnp.float32)
          + bgate_ref[0, 1])
    gw0 = 1.0 / (1.0 + jnp.exp(z1 - z0))
    gw1 = 1.0 - gw0
    y = gw0 * gat + gw1 * gcn + x_ref[...]
    mu = jnp.mean(y, axis=-1, keepdims=True)
    yc = y - mu
    var = jnp.mean(yc * yc, axis=-1, keepdims=True)
    out_ref[...] = g_ref[...] * yc * lax.rsqrt(var + 1e-5) + b_ref[...]


def _edge_pipeline(gat, tile, src_hbm, dst_hbm, tbl_hbm, asd_hbm, acc_s, den_s,
                   rows2, asrc2, adst2, exbuf2, dis_v, src2, dst2, dscat2,
                   isems, gsems, ssems):
    lanes0 = lax.iota(jnp.int32, 16)

    def full16(v):
        return jnp.full((16,), v, jnp.int32)

    def start_idx(c, p):
        off = tile * EPT + c * CHUNK
        pltpu.async_copy(src_hbm.at[pl.ds(off, CHUNK)], src2[p], isems[p])
        pltpu.async_copy(dst_hbm.at[pl.ds(off, CHUNK)], dst2[p], isems[p])

    def wait_idx(p):
        pltpu.make_async_copy(src_hbm.at[pl.ds(0, CHUNK)], src2[p], isems[p]).wait()
        pltpu.make_async_copy(dst_hbm.at[pl.ds(0, CHUNK)], dst2[p], isems[p]).wait()

    def start_gathers(p):
        pltpu.async_copy(tbl_hbm.at[src2[p]], rows2[p], gsems[p])
        if gat:
            pltpu.async_copy(asd_hbm.at[src2[p]], asrc2[p], gsems[p])
            pltpu.async_copy(asd_hbm.at[dst2[p]], adst2[p], gsems[p])

    def wait_gathers(p):
        pltpu.make_async_copy(tbl_hbm.at[src2[p]], rows2[p], gsems[p]).wait()
        if gat:
            pltpu.make_async_copy(asd_hbm.at[src2[p]], asrc2[p], gsems[p]).wait()
            pltpu.make_async_copy(asd_hbm.at[dst2[p]], adst2[p], gsems[p]).wait()

    def start_scatter(p):
        pltpu.async_copy(rows2[p], acc_s.at[dscat2[p]], ssems[p], add=True)
        if gat:
            pltpu.async_copy(exbuf2[p], den_s.at[dscat2[p]], ssems[p], add=True)

    def wait_scatter(p):
        pltpu.make_async_copy(rows2[p], acc_s.at[dscat2[p]], ssems[p]).wait()
        if gat:
            pltpu.make_async_copy(exbuf2[p], den_s.at[dscat2[p]], ssems[p]).wait()

    def compute(p):
        NG = CHUNK // 16
        for g in range(NG):
            lane = lanes0 + g * 16
            if gat:
                for h in range(H):
                    av = plsc.load_gather(asrc2[p], [lane, full16(h)])
                    bv = plsc.load_gather(adst2[p], [lane, full16(h + 4)])
                    al = av + bv
                    al = jnp.where(al > 0, al, 0.2 * al)
                    e = jnp.exp(al)
                    plsc.store_scatter(exbuf2[p], [lane, full16(h)], e)
            else:
                s16 = src2[p][pl.ds(g * 16, 16)]
                dv = plsc.load_gather(dis_v, [s16])
                plsc.store_scatter(exbuf2[p], [lane, full16(0)], dv)

        @plsc.parallel_loop(0, D)
        def _col(c):
            hidx = (c >> 5) if gat else 0
            for g in range(NG):
                lane = lanes0 + g * 16
                sc = plsc.load_gather(exbuf2[p], [lane, full16(hidx)])
                col = plsc.load_gather(rows2[p], [lane, full16(c)])
                plsc.store_scatter(rows2[p], [lane, full16(c)], col * sc)

    start_idx(0, 0)
    wait_idx(0)
    start_gathers(0)
    start_idx(1, 1)

    def body(i, carry):
        for p in (0, 1):
            ci = 2 * i + p
            if p == 0:
                pl.when(i > 0)(lambda: wait_scatter(1))
            else:
                wait_scatter(0)
            wait_idx(1 - p)
            start_gathers(1 - p)
            wait_gathers(p)
            for g in range(CHUNK // 16):
                dscat2[p][pl.ds(g * 16, 16)] = dst2[p][pl.ds(g * 16, 16)]
            compute(p)
            start_idx(ci + 2, p)
            start_scatter(p)
        return carry
    lax.fori_loop(0, NCHUNK // 2, body, 0)
    wait_gathers(0)
    wait_idx(1)
    wait_scatter(1)


def _sc_body(src_hbm, dst_hbm, asd_hbm, xw_hbm, xg_hbm, exself_hbm,
             initgat_hbm, z128_hbm, z8_hbm,
             num_out, den_out, u_out, deg_out,
             acc_s, den_s, dis_s,
             rows0, rows1, asrc0, asrc1, adst0, adst1, exbuf0, exbuf1,
             ones_v, dis_v, deg_v, dis640_v,
             src0, src1, dst0, dst1, dscat0, dscat1,
             isem0, isem1, gsem0, gsem1, ssem0, ssem1):
    core = lax.axis_index("c")
    tile = lax.axis_index("s")
    r0 = tile * NPT
    lanes0 = lax.iota(jnp.int32, 16)
    rows2, asrc2, adst2 = (rows0, rows1), (asrc0, asrc1), (adst0, adst1)
    exbuf2 = (exbuf0, exbuf1)
    src2, dst2, dscat2 = (src0, src1), (dst0, dst1), (dscat0, dscat1)
    isems, gsems, ssems = (isem0, isem1), (gsem0, gsem1), (ssem0, ssem1)

    def full16(v):
        return jnp.full((16,), v, jnp.int32)

    @pl.when(core == 0)
    def _gat():
        pltpu.sync_copy(initgat_hbm.at[pl.ds(r0, NPT)], acc_s.at[pl.ds(r0, NPT)])
        pltpu.sync_copy(exself_hbm.at[pl.ds(r0, NPT)], den_s.at[pl.ds(r0, NPT)])
        def _zero(k, carry):
            for col in range(4, 8):
                for b in exbuf2:
                    plsc.store_scatter(b, [lanes0 + k * 16, full16(col)],
                                       jnp.zeros((16,), jnp.float32))
            return carry
        lax.fori_loop(0, CHUNK // 16, _zero, 0)
        plsc.subcore_barrier()
        _edge_pipeline(True, tile, src_hbm, dst_hbm, xw_hbm, asd_hbm,
                       acc_s, den_s, rows2, asrc2, adst2, exbuf2, dis_v,
                       src2, dst2, dscat2, isems, gsems, ssems)
        plsc.subcore_barrier()
        pltpu.sync_copy(acc_s.at[pl.ds(r0, NPT)], num_out.at[pl.ds(r0, NPT)])
        pltpu.sync_copy(den_s.at[pl.ds(r0, NPT)], den_out.at[pl.ds(r0, NPT)])

    @pl.when(core == 1)
    def _gcn():
        pltpu.sync_copy(z128_hbm.at[pl.ds(r0, NPT)], acc_s.at[pl.ds(r0, NPT)])
        pltpu.sync_copy(z8_hbm.at[pl.ds(r0, NPT)], den_s.at[pl.ds(r0, NPT)])
        def _ones(k, carry):
            for col in range(8):
                plsc.store_scatter(ones_v, [lanes0 + k * 16, full16(col)],
                                   jnp.ones((16,), jnp.float32))
            return carry
        lax.fori_loop(0, CHUNK // 16, _ones, 0)
        plsc.subcore_barrier()

        def startA(c, p):
            off = tile * EPT + c * CHUNK
            pltpu.async_copy(dst_hbm.at[pl.ds(off, CHUNK)], dst2[p], isems[p])
        def waitAidx(p):
            pltpu.make_async_copy(dst_hbm.at[pl.ds(0, CHUNK)], dst2[p], isems[p]).wait()
        def waitAsc(p):
            pltpu.make_async_copy(ones_v, den_s.at[dscat2[p]], ssems[p]).wait()
        startA(0, 0)
        startA(1, 1)
        def _dbody(i, carry):
            for p in (0, 1):
                ci = 2 * i + p
                waitAidx(p)
                pl.when(i > 0)(lambda: waitAsc(p))
                for g in range(CHUNK // 16):
                    dscat2[p][pl.ds(g * 16, 16)] = dst2[p][pl.ds(g * 16, 16)]
                startA(ci + 2, p)
                pltpu.async_copy(ones_v, den_s.at[dscat2[p]], ssems[p], add=True)
            return carry
        lax.fori_loop(0, NCHUNK // 2, _dbody, 0)
        for p in (0, 1):
            waitAidx(p)
            waitAsc(p)
        plsc.subcore_barrier()
        @pl.when(tile == 0)
        def _wdeg():
            pltpu.sync_copy(den_s, deg_out)

        pltpu.sync_copy(den_s.at[pl.ds(r0, NPT)], deg_v)
        def _newton(j, carry):
            idx = lanes0 + j * 16
            dv = plsc.load_gather(deg_v, [idx, full16(0)])
            xx = dv + 1.0
            ii = plsc.bitcast(xx, jnp.int32)
            ii = jnp.int32(0x5F3759DF) - (ii >> 1)
            y = plsc.bitcast(ii, jnp.float32)
            for _ in range(3):
                y = y * (1.5 - (0.5 * xx) * (y * y))
            plsc.store_scatter(dis640_v, [idx], y)
            return carry
        lax.fori_loop(0, NPT // 16, _newton, 0)
        pltpu.sync_copy(dis640_v, dis_s.at[pl.ds(r0, NPT)])
        plsc.subcore_barrier()
        pltpu.sync_copy(dis_s, dis_v)

        _edge_pipeline(False, tile, src_hbm, dst_hbm, xg_hbm, asd_hbm,
                       acc_s, den_s, rows2, asrc2, adst2, exbuf2, dis_v,
                       src2, dst2, dscat2, isems, gsems, ssems)
        plsc.subcore_barrier()
        pltpu.sync_copy(acc_s.at[pl.ds(r0, NPT)], u_out.at[pl.ds(r0, NPT)])


@jax.jit
def kernel(x, edge_index, W_gat, att_src, att_dst, b_gat, W_gcn, b_gcn,
           W_gate, b_gate, gamma, beta):
    n = NPAD
    mask = (jnp.arange(D)[:, None] // C == jnp.arange(H)[None, :]).astype(jnp.float32)
    A_src = mask * att_src.reshape(-1)[:, None]
    A_dst = mask * att_dst.reshape(-1)[:, None]
    R = mask.T
    pad = jnp.zeros((2 * CHUNK,), jnp.int32)
    src = jnp.concatenate([edge_index[0], pad])
    dst = jnp.concatenate([edge_index[1], pad])
    x = jnp.pad(x, ((0, NPAD - N_NODES), (0, 0)))
    z128 = jnp.zeros((n, D), jnp.float32)
    z8 = jnp.zeros((n, 8), jnp.float32)

    blk = 1024
    grid = (n // blk,)
    row_spec = pl.BlockSpec((blk, D), lambda i: (i, 0))
    full = lambda s: pl.BlockSpec(s, lambda i: tuple(0 for _ in s))
    xw, asd, ex_self, init_gat, xg = pl.pallas_call(
        _tc_pre_body,
        grid=grid,
        in_specs=[row_spec, full((D, D)), full((D, H)), full((D, H)),
                  full((H, D)), full((D, D))],
        out_specs=[row_spec, pl.BlockSpec((blk, 2 * H), lambda i: (i, 0)),
                   pl.BlockSpec((blk, 8), lambda i: (i, 0)), row_spec, row_spec],
        out_shape=[jax.ShapeDtypeStruct((n, D), jnp.float32),
                   jax.ShapeDtypeStruct((n, 2 * H), jnp.float32),
                   jax.ShapeDtypeStruct((n, 8), jnp.float32),
                   jax.ShapeDtypeStruct((n, D), jnp.float32),
                   jax.ShapeDtypeStruct((n, D), jnp.float32)],
    )(x, W_gat, A_src, A_dst, R, W_gcn)

    mesh = plsc.VectorSubcoreMesh(core_axis_name="c", subcore_axis_name="s")
    sc = pl.kernel(
        _sc_body,
        out_type=[jax.ShapeDtypeStruct((n, D), jnp.float32),
                  jax.ShapeDtypeStruct((n, 8), jnp.float32),
                  jax.ShapeDtypeStruct((n, D), jnp.float32),
                  jax.ShapeDtypeStruct((n, 8), jnp.float32)],
        mesh=mesh,
        compiler_params=pltpu.CompilerParams(needs_layout_passes=False, use_tc_tiling_on_sc=False),
        scratch_types=[
            pltpu.VMEM_SHARED((NPAD, D), jnp.float32),
            pltpu.VMEM_SHARED((NPAD, 8), jnp.float32),
            pltpu.VMEM_SHARED((NPAD,), jnp.float32),
            pltpu.VMEM((CHUNK, D), jnp.float32),
            pltpu.VMEM((CHUNK, D), jnp.float32),
            pltpu.VMEM((CHUNK, 8), jnp.float32),
            pltpu.VMEM((CHUNK, 8), jnp.float32),
            pltpu.VMEM((CHUNK, 8), jnp.float32),
            pltpu.VMEM((CHUNK, 8), jnp.float32),
            pltpu.VMEM((CHUNK, 8), jnp.float32),
            pltpu.VMEM((CHUNK, 8), jnp.float32),
            pltpu.VMEM((CHUNK, 8), jnp.float32),
            pltpu.VMEM((NPAD,), jnp.float32),
            pltpu.VMEM((NPT, 8), jnp.float32),
            pltpu.VMEM((NPT,), jnp.float32),
            pltpu.VMEM((CHUNK,), jnp.int32),
            pltpu.VMEM((CHUNK,), jnp.int32),
            pltpu.VMEM((CHUNK,), jnp.int32),
            pltpu.VMEM((CHUNK,), jnp.int32),
            pltpu.VMEM((CHUNK,), jnp.int32),
            pltpu.VMEM((CHUNK,), jnp.int32),
            pltpu.SemaphoreType.DMA,
            pltpu.SemaphoreType.DMA,
            pltpu.SemaphoreType.DMA,
            pltpu.SemaphoreType.DMA,
            pltpu.SemaphoreType.DMA,
            pltpu.SemaphoreType.DMA,
        ],
    )
    num, den8, u, deg8 = sc(src, dst, asd, xw, xg, ex_self, init_gat, z128, z8)

    out = pl.pallas_call(
        _tc_post_body,
        grid=grid,
        in_specs=[row_spec, pl.BlockSpec((blk, 8), lambda i: (i, 0)), row_spec,
                  pl.BlockSpec((blk, 8), lambda i: (i, 0)), row_spec, row_spec,
                  full((H, D)), full((D, 2)), full((D, 2)), full((1, 2)),
                  full((1, D)), full((1, D)), full((1, D)), full((1, D))],
        out_specs=row_spec,
        out_shape=jax.ShapeDtypeStruct((n, D), jnp.float32),
    )(num, den8, u, deg8, xg, x, R,
      W_gate[:D], W_gate[D:], b_gate.reshape(1, 2), b_gat.reshape(1, D),
      b_gcn.reshape(1, D), gamma.reshape(1, D), beta.reshape(1, D))
    return out[:N_NODES]
